# Initial kernel scaffold; baseline (speedup 1.0000x reference)
import jax
import jax.numpy as jnp
from jax.experimental import pallas as pl


def kernel(inputs, tok_table, pos_table):
    raise NotImplementedError("write your pallas kernel here")


# speedup vs baseline: 1.4562x; 1.4562x over previous
import jax
import jax.numpy as jnp
from jax import lax
from jax.experimental import pallas as pl
from jax.experimental.pallas import tpu as pltpu
from jax.experimental.pallas import tpu_sc as plsc

SEQ = 200
DIM = 32

NC, NS = 2, 16
NW = NC * NS
CHUNK = 1024
GB = 128
NG = CHUNK // GB
POS_BIG = CHUNK + SEQ
UNROLL = 8


def _body(idx_hbm, table_hbm, pos_hbm, out_hbm,
          idx_v, dst_v, pos_v, sg0, sg1, si0, si1, so0, so1):
    rows = out_hbm.shape[0]
    rows_w = rows // NW
    nchunk = rows_w // CHUNK
    wid = lax.axis_index("s") * NC + lax.axis_index("c")
    base = wid * rows_w
    sem_g = (sg0, sg1)
    sem_i = (si0, si1)
    sem_o = (so0, so1)

    n_full = POS_BIG // SEQ
    rem = POS_BIG - n_full * SEQ
    for i in range(n_full):
        pltpu.sync_copy(pos_hbm, pos_v.at[pl.ds(i * SEQ, SEQ)])
    if rem:
        pltpu.sync_copy(pos_hbm.at[pl.ds(0, rem)], pos_v.at[pl.ds(n_full * SEQ, rem)])

    def idx_load(c, p):
        row = pl.multiple_of((base + c * CHUNK) // GB, 8)
        return pltpu.async_copy(idx_hbm.at[pl.ds(row, NG)], idx_v.at[p], sem_i[p])

    def gathers(p):
        return [
            pltpu.async_copy(
                table_hbm.at[idx_v.at[p, j]],
                dst_v.at[p, pl.ds(j * GB, GB)],
                sem_g[p],
            )
            for j in range(NG)
        ]

    def out_store(c, p):
        off = pl.multiple_of(base + c * CHUNK, 8)
        return pltpu.async_copy(dst_v.at[p], out_hbm.at[pl.ds(off, CHUNK)], sem_o[p])

    def pos_add(c, p):
        phase = (c * CHUNK) % SEQ

        def body(r, carry):
            for u in range(UNROLL):
                row = r * UNROLL + u
                lo = pos_v[phase + row, pl.ds(0, 16)]
                hi = pos_v[phase + row, pl.ds(16, 16)]
                plsc.addupdate(dst_v.at[p, row, pl.ds(0, 16)], lo)
                plsc.addupdate(dst_v.at[p, row, pl.ds(16, 16)], hi)
            return carry

        lax.fori_loop(0, CHUNK // UNROLL, body, 0)

    idx_d = [None, None]
    g_d = [None, None]
    o_d = [None, None]
    idx_load(0, 0).wait()
    g_d[0] = gathers(0)
    if nchunk > 1:
        idx_d[1] = idx_load(1, 1)
    for c in range(nchunk):
        p = c % 2
        q = 1 - p
        if c + 1 < nchunk:
            idx_d[q].wait()
            if o_d[q] is not None:
                o_d[q].wait()
            g_d[q] = gathers(q)
        for cp in g_d[p]:
            cp.wait()
        if c + 2 < nchunk:
            idx_d[p] = idx_load(c + 2, p)
        pos_add(c, p)
        o_d[p] = out_store(c, p)
    for d in o_d:
        if d is not None:
            d.wait()


def kernel(inputs, tok_table, pos_table):
    batch, seq = inputs.shape
    rows = batch * seq
    idx = inputs.reshape(rows // GB, GB).astype(jnp.int32)
    mesh = plsc.VectorSubcoreMesh(core_axis_name="c", subcore_axis_name="s")
    out = pl.kernel(
        _body,
        out_type=jax.ShapeDtypeStruct((rows, DIM), jnp.float32),
        mesh=mesh,
        compiler_params=pltpu.CompilerParams(use_tc_tiling_on_sc=False),
        scratch_types=[
            pltpu.VMEM((2, NG, GB), jnp.int32),
            pltpu.VMEM((2, CHUNK, DIM), jnp.float32),
            pltpu.VMEM((POS_BIG, DIM), jnp.float32),
            pltpu.SemaphoreType.DMA,
            pltpu.SemaphoreType.DMA,
            pltpu.SemaphoreType.DMA,
            pltpu.SemaphoreType.DMA,
            pltpu.SemaphoreType.DMA,
            pltpu.SemaphoreType.DMA,
        ],
    )(idx, tok_table, pos_table)
    return out.reshape(batch, seq, DIM)


# speedup vs baseline: 1.4913x; 1.0241x over previous
import jax
import jax.numpy as jnp
from jax import lax
from jax.experimental import pallas as pl
from jax.experimental.pallas import tpu as pltpu
from jax.experimental.pallas import tpu_sc as plsc

DIM = 32

NC, NS = 2, 16
NW = NC * NS
NB = 8
GA = 128
UNROLL = 4


def _body(idx_in, table_hbm, pos_hbm, out_hbm,
          idx_v, dst_v, pos_v, sg0, sg1, si0, si1, so0, so1):
    batch, seq = idx_in.shape
    gb = seq - GA
    batch_w = batch // NW
    nchunk = batch_w // NB
    wid = lax.axis_index("s") * NC + lax.axis_index("c")
    base = wid * batch_w
    sem_g = (sg0, sg1)
    sem_i = (si0, si1)
    sem_o = (so0, so1)

    pltpu.sync_copy(pos_hbm, pos_v)

    def idx_load(c, p):
        b0 = pl.multiple_of(base + c * NB, 8)
        return pltpu.async_copy(idx_in.at[pl.ds(b0, NB)], idx_v.at[p], sem_i[p])

    def gathers(p):
        cps = []
        for i in range(NB):
            cps.append(pltpu.async_copy(
                table_hbm.at[idx_v.at[p, i, pl.ds(0, GA)]],
                dst_v.at[p, i, pl.ds(0, GA)],
                sem_g[p],
            ))
            cps.append(pltpu.async_copy(
                table_hbm.at[idx_v.at[p, i, pl.ds(GA, gb)]],
                dst_v.at[p, i, pl.ds(GA, gb)],
                sem_g[p],
            ))
        return cps

    def out_store(c, p):
        b0 = pl.multiple_of(base + c * NB, 8)
        return pltpu.async_copy(dst_v.at[p], out_hbm.at[pl.ds(b0, NB)], sem_o[p])

    def pos_add(p):
        def body(r, carry):
            for u in range(UNROLL):
                s = r * UNROLL + u
                lo = pos_v[s, pl.ds(0, 16)]
                hi = pos_v[s, pl.ds(16, 16)]
                for i in range(NB):
                    plsc.addupdate(dst_v.at[p, i, s, pl.ds(0, 16)], lo)
                    plsc.addupdate(dst_v.at[p, i, s, pl.ds(16, 16)], hi)
            return carry

        lax.fori_loop(0, seq // UNROLL, body, 0)

    idx_d = [None, None]
    g_d = [None, None]
    o_d = [None, None]
    idx_load(0, 0).wait()
    g_d[0] = gathers(0)
    if nchunk > 1:
        idx_d[1] = idx_load(1, 1)
    for c in range(nchunk):
        p = c % 2
        q = 1 - p
        if c + 1 < nchunk:
            idx_d[q].wait()
            if o_d[q] is not None:
                o_d[q].wait()
            g_d[q] = gathers(q)
        for cp in g_d[p]:
            cp.wait()
        if c + 2 < nchunk:
            idx_d[p] = idx_load(c + 2, p)
        pos_add(p)
        o_d[p] = out_store(c, p)
    for d in o_d:
        if d is not None:
            d.wait()


def kernel(inputs, tok_table, pos_table):
    batch, seq = inputs.shape
    mesh = plsc.VectorSubcoreMesh(core_axis_name="c", subcore_axis_name="s")
    out = pl.kernel(
        _body,
        out_type=jax.ShapeDtypeStruct((batch, seq, DIM), jnp.float32),
        mesh=mesh,
        compiler_params=pltpu.CompilerParams(use_tc_tiling_on_sc=False),
        scratch_types=[
            pltpu.VMEM((2, NB, seq), jnp.int32),
            pltpu.VMEM((2, NB, seq, DIM), jnp.float32),
            pltpu.VMEM((seq, DIM), jnp.float32),
            pltpu.SemaphoreType.DMA,
            pltpu.SemaphoreType.DMA,
            pltpu.SemaphoreType.DMA,
            pltpu.SemaphoreType.DMA,
            pltpu.SemaphoreType.DMA,
            pltpu.SemaphoreType.DMA,
        ],
    )(inputs, tok_table, pos_table)
    return out
